# baseline (speedup 1.0000x reference)
import jax
import jax.numpy as jnp
from jax.experimental import pallas as pl
from jax.experimental.pallas import tpu as pltpu

D_MODEL = 4096
NUM_MODULES = 64
N_TOKENS = 8192

_WK_BLK = 512
_TOK_BLK = 512
_C = D_MODEL // _WK_BLK


def _fused_body(lt_ref, k_ref, w_ref, b_ref, e_ref, out_ref, wk_ref, bk_ref):
    j = pl.program_id(0)

    @pl.when(j < _C)
    def _precompute():
        wk_ref[:, pl.ds(j * _WK_BLK, _WK_BLK)] = jax.lax.dot_general(
            k_ref[...], w_ref[...], (((1,), (0,)), ((), ())),
            preferred_element_type=jnp.float32,
            precision=jax.lax.Precision.DEFAULT)

        @pl.when(j == 0)
        def _():
            bk_ref[...] = jnp.sum(k_ref[...] * b_ref[...], axis=1,
                                  keepdims=True).T

    @pl.when(j >= _C)
    def _route():
        logits = jax.lax.dot_general(
            e_ref[...], wk_ref[...], (((1,), (1,)), ((), ())),
            preferred_element_type=jnp.float32,
            precision=jax.lax.Precision.DEFAULT)
        temperature = jnp.maximum(jnp.exp(lt_ref[0]), 1e-4)
        inv_scale = 1.0 / ((D_MODEL ** 0.5) * temperature)
        logits = (logits + bk_ref[...]) * inv_scale
        mx = jnp.max(logits, axis=1, keepdims=True)
        ex = jnp.exp(logits - mx)
        out_ref[...] = ex / jnp.sum(ex, axis=1, keepdims=True)


def kernel(embedding, W, b, module_keys, log_temperature):
    n, d = embedding.shape
    m = module_keys.shape[0]
    t_steps = n // _TOK_BLK

    scores = pl.pallas_call(
        _fused_body,
        grid=(_C + t_steps,),
        in_specs=[
            pl.BlockSpec(memory_space=pltpu.SMEM),
            pl.BlockSpec((m, d), lambda j: (0, 0)),
            pl.BlockSpec((d, _WK_BLK), lambda j: (0, jnp.minimum(j, _C - 1))),
            pl.BlockSpec((1, d), lambda j: (0, 0)),
            pl.BlockSpec((_TOK_BLK, d), lambda j: (jnp.maximum(j - _C, 0), 0)),
        ],
        out_specs=pl.BlockSpec((_TOK_BLK, m),
                               lambda j: (jnp.maximum(j - _C, 0), 0)),
        out_shape=jax.ShapeDtypeStruct((n, m), jnp.float32),
        scratch_shapes=[
            pltpu.VMEM((m, d), jnp.float32),
            pltpu.VMEM((1, m), jnp.float32),
        ],
        compiler_params=pltpu.CompilerParams(
            dimension_semantics=("arbitrary",)),
    )(log_temperature.reshape(1), module_keys, W, b.reshape(1, d), embedding)

    return scores
